# baseline (speedup 1.0000x reference)
import functools

import jax
import jax.numpy as jnp
from jax import lax
from jax.experimental import pallas as pl
from jax.experimental.pallas import tpu as pltpu
from jax.experimental.pallas import tpu_sc as plsc

T = 64
B = 32
SPATIAL = (64, 64, 64)
N = SPATIAL[0] * SPATIAL[1] * SPATIAL[2] * 2
NC, NS, L = 2, 16, 16
NW = NC * NS
NPW = N // NW
C = 512
NSUB = NPW // C
RW = 128
KR = C // RW
ROWS = N // RW
NSL = C // L
NSTEP = T + B
SLEN = NSTEP + L
X0R = 0
ACC = 1
DB = (8, 72)
SB = (136, 168)
AROWS = 200


def _sc_body(emit_b, ts_hbm, rowb_hbm, sb_hbm, sd_hbm, x0_hbm, delta_hbm,
             out_hbm, arena, rowv, idxv, sbv, sdv, sem, semd):
    wid = lax.axis_index("s") * NC + lax.axis_index("c")

    pltpu.sync_copy(rowb_hbm, rowv)
    pltpu.sync_copy(sb_hbm, sbv)
    pltpu.sync_copy(sd_hbm, sdv)

    def dcol_copy(off, db):
        return pltpu.make_async_copy(
            delta_hbm.at[:, pl.ds(off, C)],
            arena.at[pl.ds(pl.multiple_of(db, 8), T)], semd)

    def scat_desc(k, sbase):
        return pltpu.make_async_copy(
            arena.at[pl.ds(pl.multiple_of(sbase, 8), B), pl.ds(k * RW, RW)],
            out_hbm.at[idxv.at[k]], sem)

    g0 = wid * NSUB
    for k in range(KR):
        for i in range(B // L):
            idxv[k, pl.ds(i * L, L)] = rowv[pl.ds(i * L, L)] + (g0 * KR + k)
    dcol_copy(g0 * C, DB[0]).start()
    for k in range(KR):
        scat_desc(k, SB[0]).start()

    def subchunk(sc, carry):
        par = jnp.bitwise_and(sc, 1)
        db = 8 + par * (DB[1] - DB[0])
        sbase = SB[0] + par * (SB[1] - SB[0])
        g = g0 + sc
        dcol_copy(0, DB[0]).wait()
        nxt = jnp.minimum(sc + 1, NSUB - 1)
        dcol_copy((g0 + nxt) * C, 8 + (1 - par) * (DB[1] - DB[0])).start()
        for k in range(KR):
            scat_desc(k, SB[0]).wait()
        pltpu.sync_copy(x0_hbm.at[pl.ds(g * C, C)], arena.at[X0R])
        for i in range(NSL):
            arena[ACC, pl.ds(i * L, L)] = jnp.zeros((L,), jnp.float32)
        for k in range(KR):
            for i in range(B // L):
                idxv[k, pl.ds(i * L, L)] = rowv[pl.ds(i * L, L)] + (g * KR + k)

        def step(k, cr):
            src = sbv[par, pl.ds(k, L)][0]
            dst = sdv[par, pl.ds(k, L)][0]
            for i in range(NSL):
                arena[dst, pl.ds(i * L, L)] = (
                    arena[ACC, pl.ds(i * L, L)]
                    + arena[src, pl.ds(i * L, L)])
            return cr

        lax.fori_loop(0, NSTEP, step, jnp.int32(0))

        if emit_b:
            def bsub(j, cr):
                for i in range(NSL):
                    arena[sbase + j, pl.ds(i * L, L)] = (
                        arena[sbase + j, pl.ds(i * L, L)]
                        - arena[ACC, pl.ds(i * L, L)])
                return cr

            lax.fori_loop(0, B, bsub, jnp.int32(0))

        for k in range(KR):
            scat_desc(k, sbase).start()
        return carry

    lax.fori_loop(0, NSUB, subchunk, jnp.int32(0))
    for k in range(KR):
        scat_desc(k, SB[0]).wait()
    dcol_copy(0, DB[0]).wait()


def _make_kernel(emit_b):
    return functools.partial(
        pl.kernel,
        out_type=jax.ShapeDtypeStruct((B * ROWS, RW), jnp.float32),
        scratch_types=[
            pltpu.VMEM((AROWS, C), jnp.float32),
            pltpu.VMEM((B,), jnp.int32),
            pltpu.VMEM((KR, B), jnp.int32),
            pltpu.VMEM((2, SLEN), jnp.int32),
            pltpu.VMEM((2, SLEN), jnp.int32),
            pltpu.SemaphoreType.DMA,
            pltpu.SemaphoreType.DMA,
        ],
        mesh=plsc.VectorSubcoreMesh(core_axis_name="c", subcore_axis_name="s"),
    )(functools.partial(_sc_body, emit_b))


@jax.jit
def _sc_call(ts, rowb, x0f, d2):
    jb = jnp.arange(B, dtype=jnp.int32)
    jt = jnp.arange(T, dtype=jnp.int32)
    pos_e = ts + jb
    pos_a = jt + jnp.searchsorted(ts, jt, side="right").astype(jnp.int32)

    def mk(dbase, sbase):
        sb = (jnp.zeros(SLEN, jnp.int32).at[pos_e].set(X0R)
              .at[pos_a].set(dbase + jt))
        sd = (jnp.zeros(SLEN, jnp.int32).at[pos_e].set(sbase + jb)
              .at[pos_a].set(ACC))
        return sb, sd

    sb_e, sd_e = mk(DB[0], SB[0])
    sb_o, sd_o = mk(DB[1], SB[1])
    sb2 = jnp.stack([sb_e, sb_o])
    sd2 = jnp.stack([sd_e, sd_o])

    outf = _make_kernel(False)(ts, rowb, sb2, sd2, x0f, d2)
    outb = _make_kernel(True)(ts, rowb, sb2, sd2, x0f, d2)
    return outf, outb


def kernel(slices, x0, delta):
    t_idx = slices[:, 0].astype(jnp.int32)
    order = jnp.argsort(t_idx)
    ts = t_idx[order]
    rowb = (order * ROWS).astype(jnp.int32)
    x0f = x0.reshape(N)
    d2 = delta.reshape(T, N)
    outf, outb = _sc_call(ts, rowb, x0f, d2)
    f = outf.reshape(B, *SPATIAL, 2)
    b = outb.reshape(B, *SPATIAL, 2)
    return (lax.complex(f[..., 0], f[..., 1]),
            lax.complex(b[..., 0], b[..., 1]))


# speedup vs baseline: 1.0157x; 1.0157x over previous
import functools

import jax
import jax.numpy as jnp
from jax import lax
from jax.experimental import pallas as pl
from jax.experimental.pallas import tpu as pltpu
from jax.experimental.pallas import tpu_sc as plsc

T = 64
B = 32
SPATIAL = (64, 64, 64)
N = SPATIAL[0] * SPATIAL[1] * SPATIAL[2] * 2
NC, NS, L = 2, 16, 16
NW = NC * NS
NPW = N // NW
C = 512
NSUB = NPW // C
RW = 128
KR = C // RW
ROWS = N // RW
NSL = C // L
NSTEP = T + B
SLEN = NSTEP + L
X0R = 0
ACC = 1
DBASE = 8
SBASE = DBASE + T
AROWS = SBASE + B


def _sc_body(emit_b, ts_hbm, rowb_hbm, sb_hbm, sd_hbm, x0_hbm, delta_hbm,
             out_hbm, arena, rowv, idxv, sbv, sdv, sem):
    wid = lax.axis_index("s") * NC + lax.axis_index("c")

    pltpu.sync_copy(rowb_hbm, rowv)
    pltpu.sync_copy(sb_hbm, sbv)
    pltpu.sync_copy(sd_hbm, sdv)

    def subchunk(sc, carry):
        g = wid * NSUB + sc
        off = g * C
        pltpu.sync_copy(delta_hbm.at[:, pl.ds(off, C)],
                        arena.at[pl.ds(DBASE, T)])
        pltpu.sync_copy(x0_hbm.at[pl.ds(off, C)], arena.at[X0R])
        for i in range(NSL):
            arena[ACC, pl.ds(i * L, L)] = jnp.zeros((L,), jnp.float32)
        for k in range(KR):
            for i in range(B // L):
                idxv[k, pl.ds(i * L, L)] = rowv[pl.ds(i * L, L)] + (g * KR + k)

        def step(k, cr):
            src = sbv[pl.ds(k, L)][0]
            dst = sdv[pl.ds(k, L)][0]
            for i in range(NSL):
                arena[dst, pl.ds(i * L, L)] = (
                    arena[ACC, pl.ds(i * L, L)]
                    + arena[src, pl.ds(i * L, L)])
            return cr

        lax.fori_loop(0, NSTEP, step, jnp.int32(0))

        if emit_b:
            def bsub(j, cr):
                for i in range(NSL):
                    arena[SBASE + j, pl.ds(i * L, L)] = (
                        arena[SBASE + j, pl.ds(i * L, L)]
                        - arena[ACC, pl.ds(i * L, L)])
                return cr

            lax.fori_loop(0, B, bsub, jnp.int32(0))

        cpys = [
            pltpu.make_async_copy(
                arena.at[pl.ds(SBASE, B), pl.ds(k * RW, RW)],
                out_hbm.at[idxv.at[k]], sem)
            for k in range(KR)
        ]
        for cp in cpys:
            cp.start()
        for cp in cpys:
            cp.wait()
        return carry

    lax.fori_loop(0, NSUB, subchunk, jnp.int32(0))


def _make_kernel(emit_b):
    return functools.partial(
        pl.kernel,
        out_type=jax.ShapeDtypeStruct((B * ROWS, RW), jnp.float32),
        scratch_types=[
            pltpu.VMEM((AROWS, C), jnp.float32),
            pltpu.VMEM((B,), jnp.int32),
            pltpu.VMEM((KR, B), jnp.int32),
            pltpu.VMEM((SLEN,), jnp.int32),
            pltpu.VMEM((SLEN,), jnp.int32),
            pltpu.SemaphoreType.DMA,
        ],
        mesh=plsc.VectorSubcoreMesh(core_axis_name="c", subcore_axis_name="s"),
    )(functools.partial(_sc_body, emit_b))


@jax.jit
def _sc_call(ts, rowb, x0f, d2):
    jb = jnp.arange(B, dtype=jnp.int32)
    jt = jnp.arange(T, dtype=jnp.int32)
    pos_e = ts + jb
    pos_a = jt + jnp.searchsorted(ts, jt, side="right").astype(jnp.int32)
    sb = jnp.zeros(SLEN, jnp.int32).at[pos_e].set(X0R).at[pos_a].set(DBASE + jt)
    sd = jnp.zeros(SLEN, jnp.int32).at[pos_e].set(SBASE + jb).at[pos_a].set(ACC)

    outf = _make_kernel(False)(ts, rowb, sb, sd, x0f, d2)
    outb = _make_kernel(True)(ts, rowb, sb, sd, x0f, d2)
    return outf, outb


def kernel(slices, x0, delta):
    t_idx = slices[:, 0].astype(jnp.int32)
    order = jnp.argsort(t_idx)
    ts = t_idx[order]
    rowb = (order * ROWS).astype(jnp.int32)
    x0f = x0.reshape(N)
    d2 = delta.reshape(T, N)
    outf, outb = _sc_call(ts, rowb, x0f, d2)
    f = outf.reshape(B, *SPATIAL, 2)
    b = outb.reshape(B, *SPATIAL, 2)
    return (lax.complex(f[..., 0], f[..., 1]),
            lax.complex(b[..., 0], b[..., 1]))


# speedup vs baseline: 1.0181x; 1.0024x over previous
import functools

import jax
import jax.numpy as jnp
from jax import lax
from jax.experimental import pallas as pl
from jax.experimental.pallas import tpu as pltpu
from jax.experimental.pallas import tpu_sc as plsc

T = 64
B = 32
SPATIAL = (64, 64, 64)
N = SPATIAL[0] * SPATIAL[1] * SPATIAL[2] * 2
NC, NS, L = 2, 16, 16
NW = NC * NS
NPW = N // NW
C = 512
NSUB = NPW // C
RW = 128
KR = C // RW
ROWS = N // RW
NSL = C // L
NSTEP = T + B
SLEN = NSTEP + L
X0R = 0
ACC = 1
DBASE = 8
SBASE = DBASE + T
AROWS = SBASE + B


def _sc_body(emit_b, rowb_hbm, sb_hbm, sd_hbm, x0_hbm, delta_hbm,
             out_hbm, arena, rowv, idxv, sbv, sdv, sem):
    wid = lax.axis_index("s") * NC + lax.axis_index("c")

    pltpu.sync_copy(rowb_hbm, rowv)
    pltpu.sync_copy(sb_hbm, sbv)
    pltpu.sync_copy(sd_hbm, sdv)

    def subchunk(sc, carry):
        g = wid * NSUB + sc
        off = g * C
        pltpu.sync_copy(delta_hbm.at[:, pl.ds(off, C)],
                        arena.at[pl.ds(DBASE, T)])
        pltpu.sync_copy(x0_hbm.at[pl.ds(off, C)], arena.at[X0R])
        for i in range(NSL):
            arena[ACC, pl.ds(i * L, L)] = jnp.zeros((L,), jnp.float32)
        for k in range(KR):
            for i in range(B // L):
                idxv[k, pl.ds(i * L, L)] = rowv[pl.ds(i * L, L)] + (g * KR + k)

        def step(k, cr):
            src = sbv[pl.ds(k, L)][0]
            dst = sdv[pl.ds(k, L)][0]
            for i in range(NSL):
                arena[dst, pl.ds(i * L, L)] = (
                    arena[ACC, pl.ds(i * L, L)]
                    + arena[src, pl.ds(i * L, L)])
            return cr

        lax.fori_loop(0, NSTEP, step, jnp.int32(0))

        if emit_b:
            def bsub(j, cr):
                for i in range(NSL):
                    arena[SBASE + j, pl.ds(i * L, L)] = (
                        arena[SBASE + j, pl.ds(i * L, L)]
                        - arena[ACC, pl.ds(i * L, L)])
                return cr

            lax.fori_loop(0, B, bsub, jnp.int32(0))

        cpys = [
            pltpu.make_async_copy(
                arena.at[pl.ds(SBASE, B), pl.ds(k * RW, RW)],
                out_hbm.at[idxv.at[k]], sem)
            for k in range(KR)
        ]
        for cp in cpys:
            cp.start()
        for cp in cpys:
            cp.wait()
        return carry

    lax.fori_loop(0, NSUB, subchunk, jnp.int32(0))


def _make_kernel(emit_b):
    return functools.partial(
        pl.kernel,
        out_type=jax.ShapeDtypeStruct((B * ROWS, RW), jnp.float32),
        scratch_types=[
            pltpu.VMEM((AROWS, C), jnp.float32),
            pltpu.VMEM((B,), jnp.int32),
            pltpu.VMEM((KR, B), jnp.int32),
            pltpu.VMEM((SLEN,), jnp.int32),
            pltpu.VMEM((SLEN,), jnp.int32),
            pltpu.SemaphoreType.DMA,
        ],
        mesh=plsc.VectorSubcoreMesh(core_axis_name="c", subcore_axis_name="s"),
    )(functools.partial(_sc_body, emit_b))


@jax.jit
def _sc_call(ts, rowb, x0f, d2):
    jb = jnp.arange(B, dtype=jnp.int32)
    jt = jnp.arange(T, dtype=jnp.int32)
    pos_e = ts + jb
    pos_a = jt + jnp.searchsorted(ts, jt, side="right").astype(jnp.int32)
    sb = jnp.zeros(SLEN, jnp.int32).at[pos_e].set(X0R).at[pos_a].set(DBASE + jt)
    sd = jnp.zeros(SLEN, jnp.int32).at[pos_e].set(SBASE + jb).at[pos_a].set(ACC)

    outf = _make_kernel(False)(rowb, sb, sd, x0f, d2)
    outb = _make_kernel(True)(rowb, sb, sd, x0f, d2)
    return outf, outb


def kernel(slices, x0, delta):
    t_idx = slices[:, 0].astype(jnp.int32)
    order = jnp.argsort(t_idx)
    ts = t_idx[order]
    rowb = (order * ROWS).astype(jnp.int32)
    x0f = x0.reshape(N)
    d2 = delta.reshape(T, N)
    outf, outb = _sc_call(ts, rowb, x0f, d2)
    f = outf.reshape(B, *SPATIAL, 2)
    b = outb.reshape(B, *SPATIAL, 2)
    return (lax.complex(f[..., 0], f[..., 1]),
            lax.complex(b[..., 0], b[..., 1]))
